# baseline (speedup 1.0000x reference)
import functools

import jax
import jax.numpy as jnp
from jax import lax
from jax.experimental import pallas as pl
from jax.experimental.pallas import tpu as pltpu
from jax.experimental.pallas import tpu_sc as plsc

NUM = 100000
EMB_DIM = 64
MAX_NORM = 1.0
BATCH = 4096
SEQ = 200

_INFO = plsc.get_sparse_core_info()
_NW = _INFO.num_cores * _INFO.num_subcores

_SB = 256
_NSC = BATCH // _SB
_BHL = _SB // 128
_NBH = BATCH // 128


def _renorm_body(t_ref, o_ref):
    x = t_ref[...]
    ss = jnp.sum(x * x, axis=1, keepdims=True)
    norm = jnp.sqrt(ss)
    scale = jnp.where(norm > MAX_NORM, MAX_NORM / (norm + 1e-7), 1.0)
    o_ref[...] = x * scale


def _renorm(table, block_rows):
    rows = table.shape[0]
    return pl.pallas_call(
        _renorm_body,
        grid=(rows // block_rows,),
        in_specs=[pl.BlockSpec((block_rows, EMB_DIM), lambda i: (i, 0))],
        out_specs=pl.BlockSpec((block_rows, EMB_DIM), lambda i: (i, 0)),
        out_shape=jax.ShapeDtypeStruct((rows, EMB_DIM), jnp.float32),
    )(table)


def _sc_body(xt_hbm, tab_hbm, pos_hbm, out_hbm, idx_v, tok_v, wbuf_v, pos_v, sem):
    wid = lax.axis_index("c") * _INFO.num_subcores + lax.axis_index("s")
    s_start = wid * 6 + jnp.minimum(wid, 8)
    s_count = jnp.where(wid < 8, 7, 6)
    pltpu.sync_copy(pos_hbm, pos_v)

    def pos_body(p, carry):
        @pl.when(p < s_count)
        def _():
            s = s_start + p
            pltpu.sync_copy(xt_hbm.at[s], idx_v)
            sbase = s * EMB_DIM

            def sub_body(k, carry2):
                pltpu.async_copy(
                    tab_hbm.at[idx_v.at[pl.ds(k * _SB, _SB)]], tok_v, sem
                ).wait()

                def c_body(c, carry3):
                    pv = plsc.load_gather(
                        pos_v, [jnp.full((16,), sbase + c, jnp.int32)]
                    )
                    colidx = jnp.full((16,), c, jnp.int32)
                    ch = c >> 3
                    cl = c & 7
                    for j in range(_SB // 16):
                        rows = lax.iota(jnp.int32, 16) + (16 * j)
                        v = plsc.load_gather(tok_v, [rows, colidx])
                        wbuf_v[ch, j // 8, cl, pl.ds((j % 8) * 16, 16)] = v + pv
                    return carry3

                lax.fori_loop(0, EMB_DIM, c_body, 0)
                for ch in range(8):
                    pltpu.sync_copy(
                        wbuf_v.at[ch], out_hbm.at[s, ch, pl.ds(k * _BHL, _BHL)]
                    )
                return carry2

            lax.fori_loop(0, _NSC, sub_body, 0)

        return carry

    lax.fori_loop(0, 7, pos_body, 0)


_sc_lookup = functools.partial(
    pl.kernel,
    mesh=plsc.VectorSubcoreMesh(core_axis_name="c", subcore_axis_name="s"),
    out_type=jax.ShapeDtypeStruct((SEQ, 8, _NBH, 8, 128), jnp.float32),
    scratch_types=[
        pltpu.VMEM((BATCH,), jnp.int32),
        pltpu.VMEM((_SB, EMB_DIM), jnp.float32),
        pltpu.VMEM((8, _BHL, 8, 128), jnp.float32),
        pltpu.VMEM((SEQ * EMB_DIM,), jnp.float32),
        pltpu.SemaphoreType.DMA,
    ],
    compiler_params=pltpu.CompilerParams(
        use_tc_tiling_on_sc=False, needs_layout_passes=False
    ),
)(_sc_body)


def kernel(x, token_table, pos_table):
    scaled_tab = _renorm(token_table, 1000)
    scaled_pos = _renorm(lax.slice(pos_table, (0, 0), (SEQ, EMB_DIM)), SEQ)
    xt = jnp.transpose(x.astype(jnp.int32))
    out5 = _sc_lookup(xt, scaled_tab, scaled_pos.reshape(SEQ * EMB_DIM))
    return out5.transpose(2, 4, 0, 1, 3).reshape(BATCH, SEQ, EMB_DIM)
